# Initial kernel scaffold; baseline (speedup 1.0000x reference)
import functools

import jax
import jax.numpy as jnp
from jax import lax
from jax.experimental import pallas as pl
from jax.experimental.pallas import tpu as pltpu

_TN = 512


def _chamfer_body(ppt_ref, rpt_ref, pay_ref, psr_ref, pct_ref, out_ref,
                  acc_cmin, acc_pay, acc_chx, *, nt, n, m):
    t = pl.program_id(1)

    ppt = ppt_ref[0]
    rpt = rpt_ref[0]

    ab = lax.dot_general(ppt, rpt, (((0,), (0,)), ((), ())),
                         preferred_element_type=jnp.float32)
    ones3 = jnp.ones((3, 1), jnp.float32)
    a2 = lax.dot_general(ppt * ppt, ones3, (((0,), (0,)), ((), ())),
                         preferred_element_type=jnp.float32)
    b2 = lax.dot_general(ones3, rpt * rpt, (((0,), (0,)), ((), ())),
                         preferred_element_type=jnp.float32)
    d2 = jnp.maximum(a2 + b2 - 2.0 * ab, 0.0)

    row_min = jnp.min(d2, axis=1)
    chx_part = jnp.sum(row_min)

    cmin_t = jnp.min(d2, axis=0, keepdims=True)
    iota_n = lax.broadcasted_iota(jnp.int32, (_TN, m), 0)
    cand = jnp.where(d2 == cmin_t, iota_n, jnp.int32(2**30))
    carg_t = jnp.min(cand, axis=0, keepdims=True)

    onehot = (iota_n == carg_t).astype(jnp.float32)
    pay_t = lax.dot_general(pay_ref[0], onehot, (((1,), (0,)), ((), ())),
                            preferred_element_type=jnp.float32)

    @pl.when(t == 0)
    def _init():
        acc_cmin[...] = cmin_t
        acc_pay[...] = pay_t
        acc_chx[0] = chx_part

    @pl.when(t > 0)
    def _update():
        better = cmin_t < acc_cmin[...]
        acc_cmin[...] = jnp.where(better, cmin_t, acc_cmin[...])
        acc_pay[...] = jnp.where(better, pay_t, acc_pay[...])
        acc_chx[0] = acc_chx[0] + chx_part

    @pl.when(t == nt - 1)
    def _finish():
        out_ref[0, 0] = acc_chx[0]
        out_ref[0, 1] = jnp.sum(acc_cmin[...])
        out_ref[0, 2] = jnp.sum(jnp.abs(acc_pay[0:1, :] - psr_ref[0]))
        out_ref[0, 3] = jnp.sum(jnp.abs(acc_pay[1:4, :] - pct_ref[0]))


def kernel(predicted_points, predicted_sdfs, predicted_colors, ref_points,
           ref_sdfs, ref_colors):
    pp = predicted_points.reshape(-1, *predicted_points.shape[-2:])
    ps = predicted_sdfs.reshape(-1, *predicted_sdfs.shape[-2:])
    pc = predicted_colors.reshape(-1, *predicted_colors.shape[-2:])
    rp = ref_points.reshape(-1, *ref_points.shape[-2:])
    rs = ref_sdfs.reshape(-1, *ref_sdfs.shape[-2:])
    rc = ref_colors.reshape(-1, *ref_colors.shape[-2:])

    b, n, _ = pp.shape
    m = rp.shape[1]
    nt = n // _TN

    ppt = jnp.transpose(pp, (0, 2, 1))
    rpt = jnp.transpose(rp, (0, 2, 1))
    pay = jnp.concatenate([jnp.transpose(rs, (0, 2, 1)),
                           jnp.transpose(rc, (0, 2, 1))], axis=1)
    psr = jnp.transpose(ps, (0, 2, 1))
    pct = jnp.transpose(pc, (0, 2, 1))

    body = functools.partial(_chamfer_body, nt=nt, n=n, m=m)
    sums = pl.pallas_call(
        body,
        grid=(b, nt),
        in_specs=[
            pl.BlockSpec((1, 3, _TN), lambda bb, t: (bb, 0, t)),
            pl.BlockSpec((1, 3, m), lambda bb, t: (bb, 0, 0)),
            pl.BlockSpec((1, 4, _TN), lambda bb, t: (bb, 0, t)),
            pl.BlockSpec((1, 1, m), lambda bb, t: (bb, 0, 0)),
            pl.BlockSpec((1, 3, m), lambda bb, t: (bb, 0, 0)),
        ],
        out_specs=pl.BlockSpec((1, 4), lambda bb, t: (bb, 0),
                               memory_space=pltpu.SMEM),
        out_shape=jax.ShapeDtypeStruct((b, 4), jnp.float32),
        scratch_shapes=[
            pltpu.VMEM((1, m), jnp.float32),
            pltpu.VMEM((4, m), jnp.float32),
            pltpu.SMEM((1,), jnp.float32),
        ],
    )(ppt, rpt, pay, psr, pct)

    chx = jnp.sum(sums[:, 0]) / (b * n)
    chy = jnp.sum(sums[:, 1]) / (b * m)
    sdf_l1 = jnp.sum(sums[:, 2]) / (b * m)
    color_l1 = jnp.sum(sums[:, 3]) / (b * m * 3)
    return (sdf_l1, color_l1, chx + chy)


# speedup vs baseline: 1.4534x; 1.4534x over previous
import functools

import jax
import jax.numpy as jnp
from jax import lax
from jax.experimental import pallas as pl
from jax.experimental.pallas import tpu as pltpu

_TN = 512


def _chamfer_body(ppt_ref, rpt_ref, pay_ref, psr_ref, pct_ref, out_ref,
                  acc_cmin, acc_pay, acc_chx, *, nt, n, m):
    t = pl.program_id(1)

    ppt = ppt_ref[0]
    rpt = rpt_ref[0]

    ab = lax.dot_general(ppt, rpt, (((0,), (0,)), ((), ())),
                         preferred_element_type=jnp.float32)
    ones3 = jnp.ones((3, 1), jnp.float32)
    a2 = lax.dot_general(ppt * ppt, ones3, (((0,), (0,)), ((), ())),
                         preferred_element_type=jnp.float32)
    b2 = lax.dot_general(ones3, rpt * rpt, (((0,), (0,)), ((), ())),
                         preferred_element_type=jnp.float32)
    d2 = jnp.maximum(a2 + b2 - 2.0 * ab, 0.0)

    row_min = jnp.min(d2, axis=1)
    chx_part = jnp.sum(row_min)

    cmin_t = jnp.min(d2, axis=0, keepdims=True)
    iota_n = lax.broadcasted_iota(jnp.int32, (_TN, m), 0)
    cand = jnp.where(d2 == cmin_t, iota_n, jnp.int32(2**30))
    carg_t = jnp.min(cand, axis=0, keepdims=True)

    onehot = (iota_n == carg_t).astype(jnp.float32)
    pay_t = lax.dot_general(pay_ref[0], onehot, (((1,), (0,)), ((), ())),
                            preferred_element_type=jnp.float32)

    @pl.when(t == 0)
    def _init():
        acc_cmin[...] = cmin_t
        acc_pay[...] = pay_t
        acc_chx[0] = chx_part

    @pl.when(t > 0)
    def _update():
        better = cmin_t < acc_cmin[...]
        acc_cmin[...] = jnp.where(better, cmin_t, acc_cmin[...])
        acc_pay[...] = jnp.where(better, pay_t, acc_pay[...])
        acc_chx[0] = acc_chx[0] + chx_part

    @pl.when(t == nt - 1)
    def _finish():
        bb = pl.program_id(0)
        out_ref[bb, 0] = acc_chx[0]
        out_ref[bb, 1] = jnp.sum(acc_cmin[...])
        out_ref[bb, 2] = jnp.sum(jnp.abs(acc_pay[0:1, :] - psr_ref[0]))
        out_ref[bb, 3] = jnp.sum(jnp.abs(acc_pay[1:4, :] - pct_ref[0]))


def kernel(predicted_points, predicted_sdfs, predicted_colors, ref_points,
           ref_sdfs, ref_colors):
    pp = predicted_points.reshape(-1, *predicted_points.shape[-2:])
    ps = predicted_sdfs.reshape(-1, *predicted_sdfs.shape[-2:])
    pc = predicted_colors.reshape(-1, *predicted_colors.shape[-2:])
    rp = ref_points.reshape(-1, *ref_points.shape[-2:])
    rs = ref_sdfs.reshape(-1, *ref_sdfs.shape[-2:])
    rc = ref_colors.reshape(-1, *ref_colors.shape[-2:])

    b, n, _ = pp.shape
    m = rp.shape[1]
    nt = n // _TN

    ppt = jnp.transpose(pp, (0, 2, 1))
    rpt = jnp.transpose(rp, (0, 2, 1))
    pay = jnp.concatenate([jnp.transpose(rs, (0, 2, 1)),
                           jnp.transpose(rc, (0, 2, 1))], axis=1)
    psr = jnp.transpose(ps, (0, 2, 1))
    pct = jnp.transpose(pc, (0, 2, 1))

    body = functools.partial(_chamfer_body, nt=nt, n=n, m=m)
    sums = pl.pallas_call(
        body,
        grid=(b, nt),
        in_specs=[
            pl.BlockSpec((1, 3, _TN), lambda bb, t: (bb, 0, t)),
            pl.BlockSpec((1, 3, m), lambda bb, t: (bb, 0, 0)),
            pl.BlockSpec((1, 4, _TN), lambda bb, t: (bb, 0, t)),
            pl.BlockSpec((1, 1, m), lambda bb, t: (bb, 0, 0)),
            pl.BlockSpec((1, 3, m), lambda bb, t: (bb, 0, 0)),
        ],
        out_specs=pl.BlockSpec(memory_space=pltpu.SMEM),
        out_shape=jax.ShapeDtypeStruct((b, 4), jnp.float32),
        scratch_shapes=[
            pltpu.VMEM((1, m), jnp.float32),
            pltpu.VMEM((4, m), jnp.float32),
            pltpu.SMEM((1,), jnp.float32),
        ],
    )(ppt, rpt, pay, psr, pct)

    chx = jnp.sum(sums[:, 0]) / (b * n)
    chy = jnp.sum(sums[:, 1]) / (b * m)
    sdf_l1 = jnp.sum(sums[:, 2]) / (b * m)
    color_l1 = jnp.sum(sums[:, 3]) / (b * m * 3)
    return (sdf_l1, color_l1, chx + chy)


# speedup vs baseline: 2.1763x; 1.4974x over previous
import functools

import jax
import jax.numpy as jnp
from jax import lax
from jax.experimental import pallas as pl
from jax.experimental.pallas import tpu as pltpu

_TN = 512


def _chamfer_body(ppn_ref, rpt_ref, pay_ref, psr_ref, pct_ref, out_ref,
                  acc_cmin, acc_pay, acc_chx, *, nt, n, m):
    t = pl.program_id(1)

    d2 = jnp.maximum(
        lax.dot_general(ppn_ref[0], rpt_ref[0], (((1,), (0,)), ((), ())),
                        preferred_element_type=jnp.float32), 0.0)

    row_min = jnp.min(d2, axis=1)
    chx_part = jnp.sum(row_min)

    cmin_t = jnp.min(d2, axis=0, keepdims=True)
    iota_n = lax.broadcasted_iota(jnp.int32, (_TN, m), 0)
    cand = jnp.where(d2 == cmin_t, iota_n, jnp.int32(2**30))
    carg_t = jnp.min(cand, axis=0, keepdims=True)

    onehot = (iota_n == carg_t).astype(jnp.float32)
    pay_t = lax.dot_general(pay_ref[0], onehot, (((1,), (0,)), ((), ())),
                            preferred_element_type=jnp.float32)

    @pl.when(t == 0)
    def _init():
        acc_cmin[...] = cmin_t
        acc_pay[...] = pay_t
        acc_chx[0] = chx_part

    @pl.when(t > 0)
    def _update():
        better = cmin_t < acc_cmin[...]
        acc_cmin[...] = jnp.where(better, cmin_t, acc_cmin[...])
        acc_pay[...] = jnp.where(better, pay_t, acc_pay[...])
        acc_chx[0] = acc_chx[0] + chx_part

    @pl.when(t == nt - 1)
    def _finish():
        bb = pl.program_id(0)
        out_ref[bb, 0] = acc_chx[0]
        out_ref[bb, 1] = jnp.sum(acc_cmin[...])
        out_ref[bb, 2] = jnp.sum(jnp.abs(acc_pay[0:1, :] - psr_ref[0]))
        out_ref[bb, 3] = jnp.sum(jnp.abs(acc_pay[1:4, :] - pct_ref[0]))


def kernel(predicted_points, predicted_sdfs, predicted_colors, ref_points,
           ref_sdfs, ref_colors):
    pp = predicted_points.reshape(-1, *predicted_points.shape[-2:])
    ps = predicted_sdfs.reshape(-1, *predicted_sdfs.shape[-2:])
    pc = predicted_colors.reshape(-1, *predicted_colors.shape[-2:])
    rp = ref_points.reshape(-1, *ref_points.shape[-2:])
    rs = ref_sdfs.reshape(-1, *ref_sdfs.shape[-2:])
    rc = ref_colors.reshape(-1, *ref_colors.shape[-2:])

    b, n, _ = pp.shape
    m = rp.shape[1]
    nt = n // _TN

    zeros_p = jnp.zeros((b, n, 3), jnp.float32)
    aug_p = jnp.concatenate(
        [-2.0 * pp, jnp.ones((b, n, 1), jnp.float32),
         jnp.sum(pp * pp, axis=-1, keepdims=True), zeros_p], axis=-1)
    aug_r = jnp.concatenate(
        [rp, jnp.sum(rp * rp, axis=-1, keepdims=True),
         jnp.ones((b, m, 1), jnp.float32), jnp.zeros((b, m, 3), jnp.float32)],
        axis=-1)
    aug_rt = jnp.transpose(aug_r, (0, 2, 1))
    pay = jnp.concatenate([jnp.transpose(rs, (0, 2, 1)),
                           jnp.transpose(rc, (0, 2, 1))], axis=1)
    psr = jnp.transpose(ps, (0, 2, 1))
    pct = jnp.transpose(pc, (0, 2, 1))

    body = functools.partial(_chamfer_body, nt=nt, n=n, m=m)
    sums = pl.pallas_call(
        body,
        grid=(b, nt),
        in_specs=[
            pl.BlockSpec((1, _TN, 8), lambda bb, t: (bb, t, 0)),
            pl.BlockSpec((1, 8, m), lambda bb, t: (bb, 0, 0)),
            pl.BlockSpec((1, 4, _TN), lambda bb, t: (bb, 0, t)),
            pl.BlockSpec((1, 1, m), lambda bb, t: (bb, 0, 0)),
            pl.BlockSpec((1, 3, m), lambda bb, t: (bb, 0, 0)),
        ],
        out_specs=pl.BlockSpec(memory_space=pltpu.SMEM),
        out_shape=jax.ShapeDtypeStruct((b, 4), jnp.float32),
        scratch_shapes=[
            pltpu.VMEM((1, m), jnp.float32),
            pltpu.VMEM((4, m), jnp.float32),
            pltpu.SMEM((1,), jnp.float32),
        ],
    )(aug_p, aug_rt, pay, psr, pct)

    chx = jnp.sum(sums[:, 0]) / (b * n)
    chy = jnp.sum(sums[:, 1]) / (b * m)
    sdf_l1 = jnp.sum(sums[:, 2]) / (b * m)
    color_l1 = jnp.sum(sums[:, 3]) / (b * m * 3)
    return (sdf_l1, color_l1, chx + chy)
